# Initial kernel scaffold; baseline (speedup 1.0000x reference)
import jax
import jax.numpy as jnp
from jax.experimental import pallas as pl


def kernel(target, context, W_target, W_context):
    raise NotImplementedError("write your pallas kernel here")


# speedup vs baseline: 16.2186x; 16.2186x over previous
import functools

import jax
import jax.numpy as jnp
from jax import lax
from jax.experimental import pallas as pl
from jax.experimental.pallas import tpu as pltpu
from jax.experimental.pallas import tpu_sc as plsc

_B = 16384
_K = 5
_D = 128
_NC = 2
_NS = 16
_NW = _NC * _NS
_BPW = _B // _NW
_CB = 64
_NCH = _BPW // _CB
_CK = _CB * _K
_IC = 16 * _K
_NIC = _CK // _IC
_NCHUNKS = _NW * _NCH


def _sc_body(tgt_hbm, ctx_hbm, wt_hbm, wc_hbm, out_hbm,
             idx_t, idx_c, t_v, c_v, out_v, sem):
    wid = lax.axis_index("s") * _NC + lax.axis_index("c")
    lanes = lax.iota(jnp.int32, 16)

    def chunk_body(ch, carry):
        base = wid * _BPW + ch * _CB
        pltpu.sync_copy(tgt_hbm.at[pl.ds(base, _CB)], idx_t)
        pltpu.sync_copy(
            ctx_hbm.at[pl.ds(wid * (_NCH * _NIC) + ch * _NIC, _NIC)], idx_c)
        cps = [pltpu.async_copy(wt_hbm.at[idx_t], t_v, sem)]
        for j in range(_NIC):
            cps.append(pltpu.async_copy(
                wc_hbm.at[idx_c.at[j]], c_v.at[pl.ds(j * _IC, _IC)], sem))
        for cp in cps:
            cp.wait()

        def block_body(bg, pack):
            for bi in range(16):
                b = bg * 16 + bi
                trow = [t_v[b, pl.ds(j * 16, 16)] for j in range(8)]
                for n in range(_K):
                    q = bi * _K + n
                    acc = jnp.zeros((16,), jnp.float32)
                    for j in range(8):
                        acc = acc + trow[j] * c_v[b * _K + n, pl.ds(j * 16, 16)]
                    pack = jnp.where(lanes == (q % 16), jnp.sum(acc), pack)
                    if q % 16 == 15:
                        out_v[pl.ds(bg * (16 * _K) + (q // 16) * 16, 16)] = pack
            return pack

        lax.fori_loop(0, _CB // 16, block_body, jnp.zeros((16,), jnp.float32))
        pltpu.sync_copy(out_v, out_hbm.at[wid * _NCH + ch])
        return carry

    lax.fori_loop(0, _NCH, chunk_body, 0)


@jax.jit
def _sc_call(tgt, ctx, wt, wc):
    mesh = plsc.VectorSubcoreMesh(core_axis_name="c", subcore_axis_name="s")
    run = functools.partial(
        pl.kernel,
        mesh=mesh,
        compiler_params=pltpu.CompilerParams(needs_layout_passes=False),
        out_type=jax.ShapeDtypeStruct((_NCHUNKS, _CK), jnp.float32),
        scratch_types=[
            pltpu.VMEM((_CB,), jnp.int32),
            pltpu.VMEM((_NIC, _IC), jnp.int32),
            pltpu.VMEM((_CB, _D), jnp.float32),
            pltpu.VMEM((_CK, _D), jnp.float32),
            pltpu.VMEM((_CK,), jnp.float32),
            pltpu.SemaphoreType.DMA,
        ],
    )(_sc_body)
    return run(tgt, ctx, wt, wc)


def kernel(target, context, W_target, W_context):
    b = target.shape[0]
    k = context.shape[1]
    tgt = target.reshape(b).astype(jnp.int32)
    ctx = context.reshape(b * k // _IC, _IC).astype(jnp.int32)
    out = _sc_call(tgt, ctx, W_target, W_context)
    return out.reshape(b, k)


# speedup vs baseline: 20.2725x; 1.2500x over previous
import functools

import jax
import jax.numpy as jnp
from jax import lax
from jax.experimental import pallas as pl
from jax.experimental.pallas import tpu as pltpu
from jax.experimental.pallas import tpu_sc as plsc

_B = 16384
_K = 5
_D = 128
_NC = 2
_NS = 16
_NW = _NC * _NS
_BPW = _B // _NW
_CB = 64
_NCH = _BPW // _CB
_CK = _CB * _K
_IC = 16 * _K
_NIC = _CK // _IC
_ICW = _BPW * _K // _IC
_NCHUNKS = _NW * _NCH


def _sc_body(tgt_hbm, ctx_hbm, wt_hbm, wc_hbm, out_hbm,
             idx_t, idx_c, t0, c0, o0, t1, c1, o1, sem0, sem1):
    wid = lax.axis_index("s") * _NC + lax.axis_index("c")
    lanes = lax.iota(jnp.int32, 16)

    def fire(ch, t_b, c_b, sem):
        pltpu.async_copy(wt_hbm.at[idx_t.at[pl.ds(ch * _CB, _CB)]], t_b, sem)
        for j in range(_NIC):
            pltpu.async_copy(wc_hbm.at[idx_c.at[ch * _NIC + j]],
                             c_b.at[pl.ds(j * _IC, _IC)], sem)

    def drain(t_b, c_b, sem):
        pltpu.make_async_copy(wt_hbm.at[pl.ds(0, _CB)], t_b, sem).wait()
        for j in range(_NIC):
            pltpu.make_async_copy(wc_hbm.at[pl.ds(0, _IC)],
                                  c_b.at[pl.ds(j * _IC, _IC)], sem).wait()

    def compute(ch, t_b, c_b, o_b):
        def block_body(bg, pack):
            for bi in range(16):
                b = bg * 16 + bi
                trow = [t_b[b, pl.ds(j * 16, 16)] for j in range(8)]
                for n in range(_K):
                    q = bi * _K + n
                    acc = jnp.zeros((16,), jnp.float32)
                    for j in range(8):
                        acc = acc + trow[j] * c_b[b * _K + n, pl.ds(j * 16, 16)]
                    pack = jnp.where(lanes == (q % 16), jnp.sum(acc), pack)
                    if q % 16 == 15:
                        o_b[pl.ds(bg * (16 * _K) + (q // 16) * 16, 16)] = pack
            return pack

        lax.fori_loop(0, _CB // 16, block_body, jnp.zeros((16,), jnp.float32))
        pltpu.sync_copy(o_b, out_hbm.at[wid * _NCH + ch])

    pltpu.sync_copy(tgt_hbm.at[pl.ds(wid * _BPW, _BPW)], idx_t)
    pltpu.sync_copy(ctx_hbm.at[pl.ds(wid * _ICW, _ICW)], idx_c)
    fire(0, t0, c0, sem0)

    def pair_body(g, carry):
        ch0 = 2 * g
        fire(ch0 + 1, t1, c1, sem1)
        drain(t0, c0, sem0)
        compute(ch0, t0, c0, o0)
        fire(jnp.minimum(ch0 + 2, _NCH - 1), t0, c0, sem0)
        drain(t1, c1, sem1)
        compute(ch0 + 1, t1, c1, o1)
        return carry

    lax.fori_loop(0, _NCH // 2, pair_body, 0)
    drain(t0, c0, sem0)


@jax.jit
def _sc_call(tgt, ctx, wt, wc):
    mesh = plsc.VectorSubcoreMesh(core_axis_name="c", subcore_axis_name="s")
    run = functools.partial(
        pl.kernel,
        mesh=mesh,
        compiler_params=pltpu.CompilerParams(needs_layout_passes=False),
        out_type=jax.ShapeDtypeStruct((_NCHUNKS, _CK), jnp.float32),
        scratch_types=[
            pltpu.VMEM((_BPW,), jnp.int32),
            pltpu.VMEM((_ICW, _IC), jnp.int32),
            pltpu.VMEM((_CB, _D), jnp.float32),
            pltpu.VMEM((_CK, _D), jnp.float32),
            pltpu.VMEM((_CK,), jnp.float32),
            pltpu.VMEM((_CB, _D), jnp.float32),
            pltpu.VMEM((_CK, _D), jnp.float32),
            pltpu.VMEM((_CK,), jnp.float32),
            pltpu.SemaphoreType.DMA,
            pltpu.SemaphoreType.DMA,
        ],
    )(_sc_body)
    return run(tgt, ctx, wt, wc)


def kernel(target, context, W_target, W_context):
    b = target.shape[0]
    k = context.shape[1]
    tgt = target.reshape(b).astype(jnp.int32)
    ctx = context.reshape(b * k // _IC, _IC).astype(jnp.int32)
    out = _sc_call(tgt, ctx, W_target, W_context)
    return out.reshape(b, k)
